# baseline (speedup 1.0000x reference)
import functools

import jax
import jax.numpy as jnp
from jax import lax
from jax.experimental import pallas as pl
from jax.experimental.pallas import tpu as pltpu
from jax.experimental.pallas import tpu_sc as plsc

_N = 1000
_CHUNK = 40
_NWORK = _N // _CHUNK

_mesh = plsc.VectorSubcoreMesh(core_axis_name="c", subcore_axis_name="s")


@functools.partial(
    pl.kernel,
    mesh=_mesh,
    out_type=jax.ShapeDtypeStruct((_N,), jnp.float32),
    scratch_types=[pltpu.VMEM((_CHUNK,), jnp.float32)],
)
def _sc_lookup(emb_hbm, out_hbm, buf):
    wid = lax.axis_index("s") * 2 + lax.axis_index("c")

    @pl.when(wid < _NWORK)
    def _():
        base = pl.multiple_of(wid * _CHUNK, 8)
        pltpu.sync_copy(emb_hbm.at[pl.ds(base, _CHUNK)], buf)
        pltpu.sync_copy(buf, out_hbm.at[pl.ds(base, _CHUNK)])


def kernel(ci, emb_weight):
    del ci
    return _sc_lookup(emb_weight.reshape(_N)).reshape(_N, 1)


# speedup vs baseline: 1.0929x; 1.0929x over previous
import functools

import jax
import jax.numpy as jnp
from jax import lax
from jax.experimental import pallas as pl
from jax.experimental.pallas import tpu as pltpu
from jax.experimental.pallas import tpu_sc as plsc

_N = 1000
_CHUNK = 64

_mesh = plsc.VectorSubcoreMesh(
    core_axis_name="c", subcore_axis_name="s", num_cores=1)


@functools.partial(
    pl.kernel,
    mesh=_mesh,
    out_type=jax.ShapeDtypeStruct((_N,), jnp.float32),
    scratch_types=[pltpu.VMEM((_CHUNK,), jnp.float32)],
)
def _sc_lookup(emb_hbm, out_hbm, buf):
    wid = lax.axis_index("s")

    @pl.when(wid < 15)
    def _():
        base = pl.multiple_of(wid * _CHUNK, 8)
        pltpu.sync_copy(emb_hbm.at[pl.ds(base, _CHUNK)], buf)
        pltpu.sync_copy(buf, out_hbm.at[pl.ds(base, _CHUNK)])

    @pl.when(wid == 15)
    def _():
        pltpu.sync_copy(emb_hbm.at[pl.ds(960, 40)], buf.at[pl.ds(0, 40)])
        pltpu.sync_copy(buf.at[pl.ds(0, 40)], out_hbm.at[pl.ds(960, 40)])


def kernel(ci, emb_weight):
    del ci
    return _sc_lookup(emb_weight.reshape(_N)).reshape(_N, 1)


# speedup vs baseline: 1.1592x; 1.0606x over previous
import functools

import jax
import jax.numpy as jnp
from jax.experimental import pallas as pl
from jax.experimental.pallas import tpu as pltpu
from jax.experimental.pallas import tpu_sc as plsc

_N = 1000

_mesh = plsc.ScalarSubcoreMesh(axis_name="c", num_cores=1)


@functools.partial(
    pl.kernel,
    mesh=_mesh,
    out_type=jax.ShapeDtypeStruct((_N,), jnp.float32),
    scratch_types=[pltpu.MemorySpace.VMEM_SHARED((_N,), jnp.float32)],
)
def _sc_lookup(emb_hbm, out_hbm, spmem):
    pltpu.sync_copy(emb_hbm, spmem)
    pltpu.sync_copy(spmem, out_hbm)


def kernel(ci, emb_weight):
    del ci
    return _sc_lookup(emb_weight.reshape(_N)).reshape(_N, 1)
